# baseline (speedup 1.0000x reference)
import functools

import numpy as np

import jax
import jax.numpy as jnp
from jax import lax
from jax.experimental import pallas as pl
from jax.experimental.pallas import tpu as pltpu
from jax.experimental.pallas import tpu_sc as plsc

NEG = -1e30
VERY_NEG = -3.0e38
BIG_IDX = 2 ** 30

B = 4
N = 25200
C = 85
NC = 2
NS = 16
NW = NC * NS
WPB = NW // B
SHARD = 3200
NPHYS = 25216
LASTOFF = NPHYS - SHARD
NVEC_FULL = SHARD // 16
COLS = (0, 1, 2, 3, 4, 16)
NCOL = len(COLS)
REC = 112
FB_ROW = 16040


def _iota16():
    return lax.iota(jnp.int32, 16)


def _go_bounds(praw_ref, b):
    row = praw_ref[pl.ds(b * 16, 16)]
    iota = _iota16()

    def bc(k):
        return jnp.zeros((16,), jnp.float32) + jnp.max(
            jnp.where(iota == k, row, VERY_NEG))

    f00 = bc(0)
    f20 = bc(1)
    f01 = bc(2)
    f21 = bc(3)
    s2 = bc(4)
    s3 = bc(5)
    cxL = f00 * 640.0 / s3
    cxR = f20 * 640.0 / s3
    cyU = f01 * 640.0 / s2
    cyD = f21 * 640.0 / s2
    w = cxR - cxL
    h = cyD - cyU
    cxl = cxL - 0.15 * w
    cxr = cxR + 0.15 * w
    cyu = cyU - 0.15 * h
    cyd = cyD + 0.15 * h
    return cxL, cxR, cyU, cyD, cxl, cxr, cyu, cyd


def _fused_kernel(predt_ref, praw_ref, tbl_ref, out_ref, colbuf, stp_all,
                  praw_v, rec_v, cand_sh, cand_v, tbl_v, orec_v, sem):
    cid = lax.axis_index("c")
    sid = lax.axis_index("s")
    wid = cid * NS + sid
    b = wid >> 3
    wi = wid & 7
    wstart = wi * SHARD
    off = jnp.minimum(wstart, LASTOFF)
    sub = wstart - off
    wlen = jnp.minimum(SHARD, N - wstart)
    iota = _iota16()

    pltpu.sync_copy(praw_ref, praw_v)

    copies = []
    for ci, c in enumerate(COLS):
        copies.append(pltpu.async_copy(
            predt_ref.at[c, b, pl.ds(pl.multiple_of(off, 128), SHARD)],
            colbuf.at[pl.ds(ci * SHARD, SHARD)], sem))
    for cp in copies:
        cp.wait()

    _, _, _, _, cxl, cxr, cyu, cyd = _go_bounds(praw_v, b)

    nvec = (wlen + 15) >> 4

    def vec_body(i, vo):
        pos = sub + i * 16
        cx = colbuf[pl.ds(pos, 16)]
        cy = colbuf[pl.ds(SHARD + pos, 16)]
        w_ = colbuf[pl.ds(2 * SHARD + pos, 16)]
        h_ = colbuf[pl.ds(3 * SHARD + pos, 16)]
        ob = colbuf[pl.ds(4 * SHARD + pos, 16)]
        st = colbuf[pl.ds(5 * SHARD + pos, 16)]
        m = ((cx - w_ * 0.5 > cxl) & (cx + w_ * 0.5 < cxr)
             & (cy - h_ * 0.5 > cyu) & (cy + h_ * 0.5 < cyd))
        vo = jnp.maximum(vo, jnp.where(m, ob, NEG))
        stp_all[pl.ds(i * 16, 16)] = jnp.where(m, st * ob, NEG)
        return vo

    vobj = lax.fori_loop(0, nvec, vec_body, jnp.full((16,), NEG, jnp.float32))
    obj_max = jnp.max(vobj)

    def find_max(excl1, excl2):
        def pass_body(j, carry):
            bv, bi = carry
            gi = j * 16 + iota
            v = stp_all[pl.ds(j * 16, 16)]
            v = jnp.where(gi < wlen, v, VERY_NEG)
            v = jnp.where(gi == excl1, VERY_NEG, v)
            v = jnp.where(gi == excl2, VERY_NEG, v)
            take = (v > bv) | ((v == bv) & (gi < bi))
            return jnp.where(take, v, bv), jnp.where(take, gi, bi)

        bv0 = jnp.full((16,), VERY_NEG, jnp.float32)
        bi0 = jnp.full((16,), BIG_IDX, jnp.int32)
        bv, bi = lax.fori_loop(0, NVEC_FULL, pass_body, (bv0, bi0))
        m = jnp.max(bv)
        i = jnp.min(jnp.where(bv == m, bi, BIG_IDX))
        return m, i

    v1, i1 = find_max(-1, -1)
    v2, i2 = find_max(i1, -1)
    v3, i3 = find_max(i1, i2)

    any_f = jnp.where(obj_max > -1e29, 1.0, 0.0)
    meta = (jnp.where(iota == 0, obj_max, 0.0)
            + jnp.where(iota == 1, any_f, 0.0)
            + jnp.where(iota == 2, v1, 0.0)
            + jnp.where(iota == 3, (wstart + i1).astype(jnp.float32), 0.0)
            + jnp.where(iota == 4, v2, 0.0)
            + jnp.where(iota == 5, (wstart + i2).astype(jnp.float32), 0.0)
            + jnp.where(iota == 6, v3, 0.0)
            + jnp.where(iota == 7, (wstart + i3).astype(jnp.float32), 0.0))
    rec_v[pl.ds(0, 16)] = meta

    lidx = jnp.where(iota == 0, i1,
                     jnp.where(iota == 1, i2,
                               jnp.where(iota == 2, i3,
                                         jnp.where(iota == 3, 40,
                                                   jnp.where(iota == 4, 0,
                                                             1)))))
    bidx = sub + lidx
    for ci in range(NCOL):
        rec_v[pl.ds(16 + ci * 16, 16)] = plsc.load_gather(
            colbuf, [bidx + ci * SHARD])

    pltpu.sync_copy(rec_v, cand_sh.at[pl.ds(
        pl.multiple_of(sid * REC, 8), REC)])
    plsc.subcore_barrier()

    @pl.when(sid == 0)
    def _():
        pltpu.sync_copy(tbl_ref, tbl_v)
        pltpu.sync_copy(cand_sh, cand_v)
        gcz = (cand_v[pl.ds(0, 16)] * 0.0).astype(jnp.int32)

        mcs, objs, stops = [], [], []
        for bl in range(2):
            gb = cid * 2 + bl
            cxL, cxR, cyU, cyD, _, _, _, _ = _go_bounds(praw_v, gb)
            area_go = (cxR - cxL) * (cyD - cyU)
            base = bl * WPB * REC

            halves = []
            for half in range(2):
                p = half * 16 + iota
                valid = p < 3 * WPB
                fl = tbl_v[pl.ds(half * 16, 16)] + base + gcz
                vv = plsc.load_gather(cand_v, [fl])
                iv = plsc.load_gather(cand_v, [fl + 1])
                vv = jnp.where(valid, vv, VERY_NEG)
                iv = jnp.where(valid, iv, 1.0e9)
                halves.append((vv, iv, p.astype(jnp.float32)))

            def top1(excl1, excl2, halves=halves):
                bv = jnp.full((16,), VERY_NEG, jnp.float32)
                bi = jnp.full((16,), 1.0e9, jnp.float32)
                bp = jnp.full((16,), 1.0e9, jnp.float32)
                for vv, iv, pv in halves:
                    v = jnp.where(iv == excl1, VERY_NEG, vv)
                    v = jnp.where(iv == excl2, VERY_NEG, v)
                    take = (v > bv) | ((v == bv) & (iv < bi))
                    bv = jnp.where(take, v, bv)
                    bi = jnp.where(take, iv, bi)
                    bp = jnp.where(take, pv, bp)
                m = jnp.max(bv)
                i = jnp.min(jnp.where(bv == m, bi, 1.0e9))
                p = jnp.min(jnp.where((bv == m) & (bi == i), bp, 1.0e9))
                return m, i, p

            _, i1, p1 = top1(-1.0, -1.0)
            _, i2, p2 = top1(i1, -1.0)
            _, i3, p3 = top1(i1, i2)

            rows = tbl_v[pl.ds(32, 16)] + base + gcz
            ov = plsc.load_gather(cand_v, [rows])
            obj_max = jnp.max(jnp.where(iota < WPB, ov, VERY_NEG))
            any_b = obj_max > -1e29

            pv = jnp.where(iota == 0, p1, jnp.where(iota == 1, p2, p3))
            pvi = pv.astype(jnp.int32)
            pd3 = plsc.load_gather(tbl_v, [48 + pvi + gcz])
            pm3 = plsc.load_gather(tbl_v, [80 + pvi + gcz])
            base_norm = base + pd3 * REC + 16 + pm3
            base_fb = (jnp.where(iota == 0, (5 * REC) + 16 + 3,
                                 jnp.where(iota == 1, 16 + 4, 16 + 5))
                       + base)
            basev = jnp.where(any_b, base_norm, base_fb)

            cx = plsc.load_gather(cand_v, [basev])
            cy = plsc.load_gather(cand_v, [basev + 16])
            w_ = plsc.load_gather(cand_v, [basev + 32])
            h_ = plsc.load_gather(cand_v, [basev + 48])
            ob = plsc.load_gather(cand_v, [basev + 64])
            st = plsc.load_gather(cand_v, [basev + 80])
            tv = st * ob

            x1 = cx - w_ * 0.5
            y1 = cy - h_ * 0.5
            x2 = cx + w_ * 0.5
            y2 = cy + h_ * 0.5
            area_s = (x2 - x1) * (y2 - y1)
            wx = jnp.maximum(jnp.minimum(cxR, x2) - jnp.maximum(cxL, x1), 0.0)
            wy = jnp.maximum(jnp.minimum(cyD, y2) - jnp.maximum(cyU, y1), 0.0)
            inter = wx * wy
            iouv = inter / (area_go + area_s - inter)
            lane3 = iota < 3
            iou_b = jnp.sum(jnp.where(lane3, iouv * tv, 0.0)) * (1.0 / 3.0)
            stop_b = jnp.sum(jnp.where(lane3, tv, 0.0)) * (1.0 / 3.0)
            ob0 = jnp.max(jnp.where(iota == 0, ob, VERY_NEG))
            obj_b = jnp.where(any_b, obj_max, ob0)

            mcs.append(1.0 - iou_b)
            objs.append(obj_b)
            stops.append(stop_b)
            iou_last = iou_b

        rec = (jnp.where(iota == 0, mcs[0], 0.0)
               + jnp.where(iota == 1, mcs[1], 0.0)
               + jnp.where(iota == 2, objs[0], 0.0)
               + jnp.where(iota == 3, objs[1], 0.0)
               + jnp.where(iota == 4, stops[0], 0.0)
               + jnp.where(iota == 5, stops[1], 0.0)
               + jnp.where(iota == 6, iou_last, 0.0))
        orec_v[...] = rec
        pltpu.sync_copy(orec_v, out_ref.at[pl.ds(
            pl.multiple_of(cid * 16, 8), 16)])


@functools.lru_cache(maxsize=1)
def _build():
    mesh = plsc.VectorSubcoreMesh(
        core_axis_name="c", subcore_axis_name="s",
        num_cores=NC, num_subcores=NS)
    params = pltpu.CompilerParams(needs_layout_passes=False)
    return pl.kernel(
        _fused_kernel,
        out_type=jax.ShapeDtypeStruct((NC * 16,), jnp.float32),
        mesh=mesh,
        compiler_params=params,
        scratch_types=[
            pltpu.VMEM((NCOL * SHARD,), jnp.float32),
            pltpu.VMEM((SHARD,), jnp.float32),
            pltpu.VMEM((B * 16,), jnp.float32),
            pltpu.VMEM((REC,), jnp.float32),
            pltpu.VMEM_SHARED((NS * REC,), jnp.float32),
            pltpu.VMEM((NS * REC,), jnp.float32),
            pltpu.VMEM((112,), jnp.int32),
            pltpu.VMEM((16,), jnp.float32),
            pltpu.SemaphoreType.DMA,
        ],
    )


def _tables():
    pa = np.arange(16)
    p0 = np.minimum(pa, 3 * WPB - 1)
    p1 = np.minimum(pa + 16, 3 * WPB - 1)
    pd = np.minimum(np.arange(32), 3 * WPB - 1)
    return np.concatenate([
        (p0 // 3) * REC + 2 + 2 * (p0 % 3),
        (p1 // 3) * REC + 2 + 2 * (p1 % 3),
        np.minimum(pa, WPB - 1) * REC,
        pd // 3,
        pd % 3,
    ]).astype(np.int32)


_TBL = _tables()


def kernel(predictions, coordinate_batch, car_batch, shape):
    del car_batch
    assert predictions.shape == (B, N, C)
    predt = predictions.transpose(2, 0, 1)
    sf = shape.astype(jnp.float32)
    praw = jnp.stack([
        coordinate_batch[:, 0, 0], coordinate_batch[:, 2, 0],
        coordinate_batch[:, 0, 1], coordinate_batch[:, 2, 1],
        jnp.broadcast_to(sf[2], (B,)), jnp.broadcast_to(sf[3], (B,)),
    ], axis=1)
    praw = jnp.pad(praw, ((0, 0), (0, 10))).reshape(B * 16)

    fused = _build()
    o = fused(predt, praw, jnp.asarray(_TBL))
    max_conf = jnp.stack([o[0], o[1], o[16], o[17]])
    obj_mean = (((o[2] + o[3]) + o[18]) + o[19]) * 0.25
    stop_mean = (((o[4] + o[5]) + o[20]) + o[21]) * 0.25
    return (max_conf, obj_mean, stop_mean, o[22])


# speedup vs baseline: 1.0964x; 1.0964x over previous
import functools

import numpy as np

import jax
import jax.numpy as jnp
from jax import lax
from jax.experimental import pallas as pl
from jax.experimental.pallas import tpu as pltpu
from jax.experimental.pallas import tpu_sc as plsc

NEG = -1e30
VERY_NEG = -3.0e38
BIG_IDX = 2 ** 30

B = 4
N = 25200
C = 85
NC = 2
NS = 16
NW = NC * NS
WPB = NW // B
SHARD = 3200
NPHYS = 25216
LASTOFF = NPHYS - SHARD
NVEC_FULL = SHARD // 16
COLS = (0, 1, 2, 3, 4, 16)
NCOL = len(COLS)
REC = 112
FB_ROW = 16040


def _iota16():
    return lax.iota(jnp.int32, 16)


def _go_bounds(praw_ref, b):
    row = praw_ref[pl.ds(b * 16, 16)]
    iota = _iota16()

    def bc(k):
        return jnp.zeros((16,), jnp.float32) + jnp.max(
            jnp.where(iota == k, row, VERY_NEG))

    f00 = bc(0)
    f20 = bc(1)
    f01 = bc(2)
    f21 = bc(3)
    s2 = bc(4)
    s3 = bc(5)
    cxL = f00 * 640.0 / s3
    cxR = f20 * 640.0 / s3
    cyU = f01 * 640.0 / s2
    cyD = f21 * 640.0 / s2
    w = cxR - cxL
    h = cyD - cyU
    cxl = cxL - 0.15 * w
    cxr = cxR + 0.15 * w
    cyu = cyU - 0.15 * h
    cyd = cyD + 0.15 * h
    return cxL, cxR, cyU, cyD, cxl, cxr, cyu, cyd


def _scan_kernel(predt_ref, praw_ref, cand_ref, colbuf, stp_all,
                 praw_v, rec_v, sem):
    wid = lax.axis_index("c") * NS + lax.axis_index("s")
    b = wid >> 3
    wi = wid & 7
    wstart = wi * SHARD
    off = jnp.minimum(wstart, LASTOFF)
    sub = wstart - off
    wlen = jnp.minimum(SHARD, N - wstart)
    iota = _iota16()

    pltpu.sync_copy(praw_ref, praw_v)

    copies = []
    for ci, c in enumerate(COLS):
        copies.append(pltpu.async_copy(
            predt_ref.at[c, b, pl.ds(pl.multiple_of(off, 128), SHARD)],
            colbuf.at[pl.ds(ci * SHARD, SHARD)], sem))
    for cp in copies:
        cp.wait()

    _, _, _, _, cxl, cxr, cyu, cyd = _go_bounds(praw_v, b)

    nvec = (wlen + 15) >> 4

    def vec_body(i, vo):
        pos = sub + i * 16
        cx = colbuf[pl.ds(pos, 16)]
        cy = colbuf[pl.ds(SHARD + pos, 16)]
        w_ = colbuf[pl.ds(2 * SHARD + pos, 16)]
        h_ = colbuf[pl.ds(3 * SHARD + pos, 16)]
        ob = colbuf[pl.ds(4 * SHARD + pos, 16)]
        st = colbuf[pl.ds(5 * SHARD + pos, 16)]
        m = ((cx - w_ * 0.5 > cxl) & (cx + w_ * 0.5 < cxr)
             & (cy - h_ * 0.5 > cyu) & (cy + h_ * 0.5 < cyd))
        vo = jnp.maximum(vo, jnp.where(m, ob, NEG))
        stp_all[pl.ds(i * 16, 16)] = jnp.where(m, st * ob, NEG)
        return vo

    vobj = lax.fori_loop(0, nvec, vec_body, jnp.full((16,), NEG, jnp.float32))
    obj_max = jnp.max(vobj)

    def find_max(excl1, excl2):
        def pass_body(j, carry):
            bv, bi = carry
            gi = j * 16 + iota
            v = stp_all[pl.ds(j * 16, 16)]
            v = jnp.where(gi < wlen, v, VERY_NEG)
            v = jnp.where(gi == excl1, VERY_NEG, v)
            v = jnp.where(gi == excl2, VERY_NEG, v)
            take = (v > bv) | ((v == bv) & (gi < bi))
            return jnp.where(take, v, bv), jnp.where(take, gi, bi)

        bv0 = jnp.full((16,), VERY_NEG, jnp.float32)
        bi0 = jnp.full((16,), BIG_IDX, jnp.int32)
        bv, bi = lax.fori_loop(0, NVEC_FULL, pass_body, (bv0, bi0))
        m = jnp.max(bv)
        i = jnp.min(jnp.where(bv == m, bi, BIG_IDX))
        return m, i

    v1, i1 = find_max(-1, -1)
    v2, i2 = find_max(i1, -1)
    v3, i3 = find_max(i1, i2)

    any_f = jnp.where(obj_max > -1e29, 1.0, 0.0)
    meta = (jnp.where(iota == 0, obj_max, 0.0)
            + jnp.where(iota == 1, any_f, 0.0)
            + jnp.where(iota == 2, v1, 0.0)
            + jnp.where(iota == 3, (wstart + i1).astype(jnp.float32), 0.0)
            + jnp.where(iota == 4, v2, 0.0)
            + jnp.where(iota == 5, (wstart + i2).astype(jnp.float32), 0.0)
            + jnp.where(iota == 6, v3, 0.0)
            + jnp.where(iota == 7, (wstart + i3).astype(jnp.float32), 0.0))
    rec_v[pl.ds(0, 16)] = meta

    lidx = jnp.where(iota == 0, i1,
                     jnp.where(iota == 1, i2,
                               jnp.where(iota == 2, i3,
                                         jnp.where(iota == 3, 40,
                                                   jnp.where(iota == 4, 0,
                                                             1)))))
    bidx = sub + lidx
    for ci in range(NCOL):
        rec_v[pl.ds(16 + ci * 16, 16)] = plsc.load_gather(
            colbuf, [bidx + ci * SHARD])

    pltpu.sync_copy(rec_v, cand_ref.at[pl.ds(
        pl.multiple_of(wid * REC, 8), REC)])


def _merge_kernel(praw_ref, cand_hbm, tbl_ref, out_ref,
                  praw_v, cand_v, tbl_v, rec_v):
    wid = lax.axis_index("c") * NS + lax.axis_index("s")
    iota = _iota16()

    @pl.when(wid == 0)
    def _():
        pltpu.sync_copy(praw_ref, praw_v)
        pltpu.sync_copy(cand_hbm, cand_v)
        pltpu.sync_copy(tbl_ref, tbl_v)
        gcz = (cand_v[pl.ds(0, 16)] * 0.0).astype(jnp.int32)

        mcs, objs, stops = [], [], []
        iou_last = None
        for b in range(B):
            cxL, cxR, cyU, cyD, _, _, _, _ = _go_bounds(praw_v, b)
            area_go = (cxR - cxL) * (cyD - cyU)
            base = b * WPB * REC

            halves = []
            for half in range(2):
                p = half * 16 + iota
                valid = p < 3 * WPB
                fl = tbl_v[pl.ds(half * 16, 16)] + base + gcz
                vv = plsc.load_gather(cand_v, [fl])
                iv = plsc.load_gather(cand_v, [fl + 1])
                vv = jnp.where(valid, vv, VERY_NEG)
                iv = jnp.where(valid, iv, 1.0e9)
                halves.append((vv, iv, p.astype(jnp.float32)))

            def top1(excl1, excl2, halves=halves):
                bv = jnp.full((16,), VERY_NEG, jnp.float32)
                bi = jnp.full((16,), 1.0e9, jnp.float32)
                bp = jnp.full((16,), 1.0e9, jnp.float32)
                for vv, iv, pv in halves:
                    v = jnp.where(iv == excl1, VERY_NEG, vv)
                    v = jnp.where(iv == excl2, VERY_NEG, v)
                    take = (v > bv) | ((v == bv) & (iv < bi))
                    bv = jnp.where(take, v, bv)
                    bi = jnp.where(take, iv, bi)
                    bp = jnp.where(take, pv, bp)
                m = jnp.max(bv)
                i = jnp.min(jnp.where(bv == m, bi, 1.0e9))
                p = jnp.min(jnp.where((bv == m) & (bi == i), bp, 1.0e9))
                return m, i, p

            _, i1, p1 = top1(-1.0, -1.0)
            _, i2, p2 = top1(i1, -1.0)
            _, i3, p3 = top1(i1, i2)

            rows = tbl_v[pl.ds(32, 16)] + base + gcz
            ov = plsc.load_gather(cand_v, [rows])
            obj_max = jnp.max(jnp.where(iota < WPB, ov, VERY_NEG))
            any_b = obj_max > -1e29

            pv = jnp.where(iota == 0, p1, jnp.where(iota == 1, p2, p3))
            pvi = pv.astype(jnp.int32)
            pd3 = plsc.load_gather(tbl_v, [48 + pvi + gcz])
            pm3 = plsc.load_gather(tbl_v, [80 + pvi + gcz])
            base_norm = base + pd3 * REC + 16 + pm3
            base_fb = (jnp.where(iota == 0, (5 * REC) + 16 + 3,
                                 jnp.where(iota == 1, 16 + 4, 16 + 5))
                       + base)
            basev = jnp.where(any_b, base_norm, base_fb)

            cx = plsc.load_gather(cand_v, [basev])
            cy = plsc.load_gather(cand_v, [basev + 16])
            w_ = plsc.load_gather(cand_v, [basev + 32])
            h_ = plsc.load_gather(cand_v, [basev + 48])
            ob = plsc.load_gather(cand_v, [basev + 64])
            st = plsc.load_gather(cand_v, [basev + 80])
            tv = st * ob

            x1 = cx - w_ * 0.5
            y1 = cy - h_ * 0.5
            x2 = cx + w_ * 0.5
            y2 = cy + h_ * 0.5
            area_s = (x2 - x1) * (y2 - y1)
            wx = jnp.maximum(jnp.minimum(cxR, x2) - jnp.maximum(cxL, x1), 0.0)
            wy = jnp.maximum(jnp.minimum(cyD, y2) - jnp.maximum(cyU, y1), 0.0)
            inter = wx * wy
            iouv = inter / (area_go + area_s - inter)
            lane3 = iota < 3
            iou_b = jnp.sum(jnp.where(lane3, iouv * tv, 0.0)) * (1.0 / 3.0)
            stop_b = jnp.sum(jnp.where(lane3, tv, 0.0)) * (1.0 / 3.0)
            ob0 = jnp.max(jnp.where(iota == 0, ob, VERY_NEG))
            obj_b = jnp.where(any_b, obj_max, ob0)

            mcs.append(1.0 - iou_b)
            objs.append(obj_b)
            stops.append(stop_b)
            iou_last = iou_b

        obj_mean = (((objs[0] + objs[1]) + objs[2]) + objs[3]) * 0.25
        stop_mean = (((stops[0] + stops[1]) + stops[2]) + stops[3]) * 0.25
        rec = (jnp.where(iota == 0, mcs[0], 0.0)
               + jnp.where(iota == 1, mcs[1], 0.0)
               + jnp.where(iota == 2, mcs[2], 0.0)
               + jnp.where(iota == 3, mcs[3], 0.0)
               + jnp.where(iota == 4, obj_mean, 0.0)
               + jnp.where(iota == 5, stop_mean, 0.0)
               + jnp.where(iota == 6, iou_last, 0.0))
        rec_v[...] = rec
        pltpu.sync_copy(rec_v, out_ref)


@functools.lru_cache(maxsize=1)
def _build():
    mesh = plsc.VectorSubcoreMesh(
        core_axis_name="c", subcore_axis_name="s",
        num_cores=NC, num_subcores=NS)
    mesh1 = plsc.VectorSubcoreMesh(
        core_axis_name="c", subcore_axis_name="s",
        num_cores=1, num_subcores=NS)
    params = pltpu.CompilerParams(needs_layout_passes=False)
    scan = pl.kernel(
        _scan_kernel,
        out_type=jax.ShapeDtypeStruct((NW * REC,), jnp.float32),
        mesh=mesh,
        compiler_params=params,
        scratch_types=[
            pltpu.VMEM((NCOL * SHARD,), jnp.float32),
            pltpu.VMEM((SHARD,), jnp.float32),
            pltpu.VMEM((B * 16,), jnp.float32),
            pltpu.VMEM((REC,), jnp.float32),
            pltpu.SemaphoreType.DMA,
        ],
    )
    merge = pl.kernel(
        _merge_kernel,
        out_type=jax.ShapeDtypeStruct((16,), jnp.float32),
        mesh=mesh1,
        compiler_params=params,
        scratch_types=[
            pltpu.VMEM((B * 16,), jnp.float32),
            pltpu.VMEM((NW * REC,), jnp.float32),
            pltpu.VMEM((112,), jnp.int32),
            pltpu.VMEM((16,), jnp.float32),
        ],
    )
    return scan, merge


def _tables():
    pa = np.arange(16)
    p0 = np.minimum(pa, 3 * WPB - 1)
    p1 = np.minimum(pa + 16, 3 * WPB - 1)
    pd = np.minimum(np.arange(32), 3 * WPB - 1)
    return np.concatenate([
        (p0 // 3) * REC + 2 + 2 * (p0 % 3),
        (p1 // 3) * REC + 2 + 2 * (p1 % 3),
        np.minimum(pa, WPB - 1) * REC,
        pd // 3,
        pd % 3,
    ]).astype(np.int32)


_TBL = _tables()


def kernel(predictions, coordinate_batch, car_batch, shape):
    del car_batch
    assert predictions.shape == (B, N, C)
    predt = predictions.transpose(2, 0, 1)
    sf = shape.astype(jnp.float32)
    praw = jnp.stack([
        coordinate_batch[:, 0, 0], coordinate_batch[:, 2, 0],
        coordinate_batch[:, 0, 1], coordinate_batch[:, 2, 1],
        jnp.broadcast_to(sf[2], (B,)), jnp.broadcast_to(sf[3], (B,)),
    ], axis=1)
    praw = jnp.pad(praw, ((0, 0), (0, 10))).reshape(B * 16)

    scan, merge = _build()
    cand = scan(predt, praw)
    out16 = merge(praw, cand, jnp.asarray(_TBL))
    return (out16[:4], out16[4], out16[5], out16[6])


# speedup vs baseline: 1.1644x; 1.0621x over previous
import functools

import numpy as np

import jax
import jax.numpy as jnp
from jax import lax
from jax.experimental import pallas as pl
from jax.experimental.pallas import tpu as pltpu
from jax.experimental.pallas import tpu_sc as plsc

NEG = -1e30
VERY_NEG = -3.0e38
BIG_IDX = 2 ** 30

B = 4
N = 25200
C = 85
NC = 2
NS = 16
NW = NC * NS
WPB = NW // B
SHARD = 3200
NPHYS = 25216
LASTOFF = NPHYS - SHARD
NVEC_FULL = SHARD // 16
COLS = (0, 1, 2, 3, 4, 16)
NCOL = len(COLS)
REC = 112
FB_ROW = 16040


def _iota16():
    return lax.iota(jnp.int32, 16)


def _go_bounds(praw_ref, b):
    row = praw_ref[pl.ds(b * 16, 16)]
    iota = _iota16()

    def bc(k):
        return jnp.zeros((16,), jnp.float32) + jnp.max(
            jnp.where(iota == k, row, VERY_NEG))

    f00 = bc(0)
    f20 = bc(1)
    f01 = bc(2)
    f21 = bc(3)
    s2 = bc(4)
    s3 = bc(5)
    cxL = f00 * 640.0 / s3
    cxR = f20 * 640.0 / s3
    cyU = f01 * 640.0 / s2
    cyD = f21 * 640.0 / s2
    w = cxR - cxL
    h = cyD - cyU
    cxl = cxL - 0.15 * w
    cxr = cxR + 0.15 * w
    cyu = cyU - 0.15 * h
    cyd = cyD + 0.15 * h
    return cxL, cxR, cyU, cyD, cxl, cxr, cyu, cyd


def _scan_kernel(predt_ref, praw_ref, cand_ref, colbuf,
                 praw_v, rec_v, sem):
    wid = lax.axis_index("c") * NS + lax.axis_index("s")
    b = wid >> 3
    wi = wid & 7
    wstart = wi * SHARD
    off = jnp.minimum(wstart, LASTOFF)
    sub = wstart - off
    wlen = jnp.minimum(SHARD, N - wstart)
    iota = _iota16()

    pltpu.sync_copy(praw_ref, praw_v)

    copies = []
    for ci, c in enumerate(COLS):
        copies.append(pltpu.async_copy(
            predt_ref.at[c, b, pl.ds(pl.multiple_of(off, 128), SHARD)],
            colbuf.at[pl.ds(ci * SHARD, SHARD)], sem))
    for cp in copies:
        cp.wait()

    _, _, _, _, cxl, cxr, cyu, cyd = _go_bounds(praw_v, b)

    nvec = (wlen + 15) >> 4

    def vec_body(i, carry):
        vo, b1, b2, b3, j1, j2, j3 = carry
        pos = sub + i * 16
        cx = colbuf[pl.ds(pos, 16)]
        cy = colbuf[pl.ds(SHARD + pos, 16)]
        w_ = colbuf[pl.ds(2 * SHARD + pos, 16)]
        h_ = colbuf[pl.ds(3 * SHARD + pos, 16)]
        ob = colbuf[pl.ds(4 * SHARD + pos, 16)]
        st = colbuf[pl.ds(5 * SHARD + pos, 16)]
        m = ((cx - w_ * 0.5 > cxl) & (cx + w_ * 0.5 < cxr)
             & (cy - h_ * 0.5 > cyu) & (cy + h_ * 0.5 < cyd))
        vo = jnp.maximum(vo, jnp.where(m, ob, NEG))
        s = jnp.where(m, st * ob, NEG)
        gi = i * 16 + iota
        t1 = s > b1
        t2 = (~t1) & (s > b2)
        t3 = (~t1) & (~t2) & (s > b3)
        b3 = jnp.where(t1 | t2, b2, jnp.where(t3, s, b3))
        j3 = jnp.where(t1 | t2, j2, jnp.where(t3, gi, j3))
        b2 = jnp.where(t1, b1, jnp.where(t2, s, b2))
        j2 = jnp.where(t1, j1, jnp.where(t2, gi, j2))
        b1 = jnp.where(t1, s, b1)
        j1 = jnp.where(t1, gi, j1)
        return vo, b1, b2, b3, j1, j2, j3

    fneg = jnp.full((16,), VERY_NEG, jnp.float32)
    fbig = jnp.full((16,), BIG_IDX, jnp.int32)
    vobj, b1, b2, b3, j1, j2, j3 = lax.fori_loop(
        0, nvec, vec_body,
        (jnp.full((16,), NEG, jnp.float32), fneg, fneg, fneg,
         fbig, fbig, fbig))
    obj_max = jnp.max(vobj)

    def pick(b1, b2, b3, j1, j2, j3):
        m = jnp.max(b1)
        m = jnp.maximum(m, jnp.maximum(jnp.max(b2), jnp.max(b3)))
        i = jnp.minimum(
            jnp.minimum(jnp.min(jnp.where(b1 == m, j1, BIG_IDX)),
                        jnp.min(jnp.where(b2 == m, j2, BIG_IDX))),
            jnp.min(jnp.where(b3 == m, j3, BIG_IDX)))
        return m, i

    def excl(b, j, i):
        return jnp.where(j == i, VERY_NEG, b)

    v1, i1 = pick(b1, b2, b3, j1, j2, j3)
    b1e, b2e, b3e = excl(b1, j1, i1), excl(b2, j2, i1), excl(b3, j3, i1)
    v2, i2 = pick(b1e, b2e, b3e, j1, j2, j3)
    b1e, b2e, b3e = excl(b1e, j1, i2), excl(b2e, j2, i2), excl(b3e, j3, i2)
    v3, i3 = pick(b1e, b2e, b3e, j1, j2, j3)

    any_f = jnp.where(obj_max > -1e29, 1.0, 0.0)
    meta = (jnp.where(iota == 0, obj_max, 0.0)
            + jnp.where(iota == 1, any_f, 0.0)
            + jnp.where(iota == 2, v1, 0.0)
            + jnp.where(iota == 3, (wstart + i1).astype(jnp.float32), 0.0)
            + jnp.where(iota == 4, v2, 0.0)
            + jnp.where(iota == 5, (wstart + i2).astype(jnp.float32), 0.0)
            + jnp.where(iota == 6, v3, 0.0)
            + jnp.where(iota == 7, (wstart + i3).astype(jnp.float32), 0.0))
    rec_v[pl.ds(0, 16)] = meta

    lidx = jnp.where(iota == 0, i1,
                     jnp.where(iota == 1, i2,
                               jnp.where(iota == 2, i3,
                                         jnp.where(iota == 3, 40,
                                                   jnp.where(iota == 4, 0,
                                                             1)))))
    bidx = sub + lidx
    for ci in range(NCOL):
        rec_v[pl.ds(16 + ci * 16, 16)] = plsc.load_gather(
            colbuf, [bidx + ci * SHARD])

    pltpu.sync_copy(rec_v, cand_ref.at[pl.ds(
        pl.multiple_of(wid * REC, 8), REC)])


def _merge_kernel(praw_ref, cand_hbm, tbl_ref, out_ref,
                  praw_v, cand_v, tbl_v, rec_v):
    wid = lax.axis_index("c") * NS + lax.axis_index("s")
    iota = _iota16()

    @pl.when(wid == 0)
    def _():
        pltpu.sync_copy(praw_ref, praw_v)
        pltpu.sync_copy(cand_hbm, cand_v)
        pltpu.sync_copy(tbl_ref, tbl_v)
        gcz = (cand_v[pl.ds(0, 16)] * 0.0).astype(jnp.int32)

        mcs, objs, stops = [], [], []
        iou_last = None
        for b in range(B):
            cxL, cxR, cyU, cyD, _, _, _, _ = _go_bounds(praw_v, b)
            area_go = (cxR - cxL) * (cyD - cyU)
            base = b * WPB * REC

            halves = []
            for half in range(2):
                p = half * 16 + iota
                valid = p < 3 * WPB
                fl = tbl_v[pl.ds(half * 16, 16)] + base + gcz
                vv = plsc.load_gather(cand_v, [fl])
                iv = plsc.load_gather(cand_v, [fl + 1])
                vv = jnp.where(valid, vv, VERY_NEG)
                iv = jnp.where(valid, iv, 1.0e9)
                halves.append((vv, iv, p.astype(jnp.float32)))

            def top1(excl1, excl2, halves=halves):
                bv = jnp.full((16,), VERY_NEG, jnp.float32)
                bi = jnp.full((16,), 1.0e9, jnp.float32)
                bp = jnp.full((16,), 1.0e9, jnp.float32)
                for vv, iv, pv in halves:
                    v = jnp.where(iv == excl1, VERY_NEG, vv)
                    v = jnp.where(iv == excl2, VERY_NEG, v)
                    take = (v > bv) | ((v == bv) & (iv < bi))
                    bv = jnp.where(take, v, bv)
                    bi = jnp.where(take, iv, bi)
                    bp = jnp.where(take, pv, bp)
                m = jnp.max(bv)
                i = jnp.min(jnp.where(bv == m, bi, 1.0e9))
                p = jnp.min(jnp.where((bv == m) & (bi == i), bp, 1.0e9))
                return m, i, p

            _, i1, p1 = top1(-1.0, -1.0)
            _, i2, p2 = top1(i1, -1.0)
            _, i3, p3 = top1(i1, i2)

            rows = tbl_v[pl.ds(32, 16)] + base + gcz
            ov = plsc.load_gather(cand_v, [rows])
            obj_max = jnp.max(jnp.where(iota < WPB, ov, VERY_NEG))
            any_b = obj_max > -1e29

            pv = jnp.where(iota == 0, p1, jnp.where(iota == 1, p2, p3))
            pvi = pv.astype(jnp.int32)
            pd3 = plsc.load_gather(tbl_v, [48 + pvi + gcz])
            pm3 = plsc.load_gather(tbl_v, [80 + pvi + gcz])
            base_norm = base + pd3 * REC + 16 + pm3
            base_fb = (jnp.where(iota == 0, (5 * REC) + 16 + 3,
                                 jnp.where(iota == 1, 16 + 4, 16 + 5))
                       + base)
            basev = jnp.where(any_b, base_norm, base_fb)

            cx = plsc.load_gather(cand_v, [basev])
            cy = plsc.load_gather(cand_v, [basev + 16])
            w_ = plsc.load_gather(cand_v, [basev + 32])
            h_ = plsc.load_gather(cand_v, [basev + 48])
            ob = plsc.load_gather(cand_v, [basev + 64])
            st = plsc.load_gather(cand_v, [basev + 80])
            tv = st * ob

            x1 = cx - w_ * 0.5
            y1 = cy - h_ * 0.5
            x2 = cx + w_ * 0.5
            y2 = cy + h_ * 0.5
            area_s = (x2 - x1) * (y2 - y1)
            wx = jnp.maximum(jnp.minimum(cxR, x2) - jnp.maximum(cxL, x1), 0.0)
            wy = jnp.maximum(jnp.minimum(cyD, y2) - jnp.maximum(cyU, y1), 0.0)
            inter = wx * wy
            iouv = inter / (area_go + area_s - inter)
            lane3 = iota < 3
            iou_b = jnp.sum(jnp.where(lane3, iouv * tv, 0.0)) * (1.0 / 3.0)
            stop_b = jnp.sum(jnp.where(lane3, tv, 0.0)) * (1.0 / 3.0)
            ob0 = jnp.max(jnp.where(iota == 0, ob, VERY_NEG))
            obj_b = jnp.where(any_b, obj_max, ob0)

            mcs.append(1.0 - iou_b)
            objs.append(obj_b)
            stops.append(stop_b)
            iou_last = iou_b

        obj_mean = (((objs[0] + objs[1]) + objs[2]) + objs[3]) * 0.25
        stop_mean = (((stops[0] + stops[1]) + stops[2]) + stops[3]) * 0.25
        rec = (jnp.where(iota == 0, mcs[0], 0.0)
               + jnp.where(iota == 1, mcs[1], 0.0)
               + jnp.where(iota == 2, mcs[2], 0.0)
               + jnp.where(iota == 3, mcs[3], 0.0)
               + jnp.where(iota == 4, obj_mean, 0.0)
               + jnp.where(iota == 5, stop_mean, 0.0)
               + jnp.where(iota == 6, iou_last, 0.0))
        rec_v[...] = rec
        pltpu.sync_copy(rec_v, out_ref)


@functools.lru_cache(maxsize=1)
def _build():
    mesh = plsc.VectorSubcoreMesh(
        core_axis_name="c", subcore_axis_name="s",
        num_cores=NC, num_subcores=NS)
    mesh1 = plsc.VectorSubcoreMesh(
        core_axis_name="c", subcore_axis_name="s",
        num_cores=1, num_subcores=NS)
    params = pltpu.CompilerParams(needs_layout_passes=False)
    scan = pl.kernel(
        _scan_kernel,
        out_type=jax.ShapeDtypeStruct((NW * REC,), jnp.float32),
        mesh=mesh,
        compiler_params=params,
        scratch_types=[
            pltpu.VMEM((NCOL * SHARD,), jnp.float32),
            pltpu.VMEM((B * 16,), jnp.float32),
            pltpu.VMEM((REC,), jnp.float32),
            pltpu.SemaphoreType.DMA,
        ],
    )
    merge = pl.kernel(
        _merge_kernel,
        out_type=jax.ShapeDtypeStruct((16,), jnp.float32),
        mesh=mesh1,
        compiler_params=params,
        scratch_types=[
            pltpu.VMEM((B * 16,), jnp.float32),
            pltpu.VMEM((NW * REC,), jnp.float32),
            pltpu.VMEM((112,), jnp.int32),
            pltpu.VMEM((16,), jnp.float32),
        ],
    )
    return scan, merge


def _tables():
    pa = np.arange(16)
    p0 = np.minimum(pa, 3 * WPB - 1)
    p1 = np.minimum(pa + 16, 3 * WPB - 1)
    pd = np.minimum(np.arange(32), 3 * WPB - 1)
    return np.concatenate([
        (p0 // 3) * REC + 2 + 2 * (p0 % 3),
        (p1 // 3) * REC + 2 + 2 * (p1 % 3),
        np.minimum(pa, WPB - 1) * REC,
        pd // 3,
        pd % 3,
    ]).astype(np.int32)


_TBL = _tables()


def kernel(predictions, coordinate_batch, car_batch, shape):
    del car_batch
    assert predictions.shape == (B, N, C)
    predt = predictions.transpose(2, 0, 1)
    sf = shape.astype(jnp.float32)
    praw = jnp.stack([
        coordinate_batch[:, 0, 0], coordinate_batch[:, 2, 0],
        coordinate_batch[:, 0, 1], coordinate_batch[:, 2, 1],
        jnp.broadcast_to(sf[2], (B,)), jnp.broadcast_to(sf[3], (B,)),
    ], axis=1)
    praw = jnp.pad(praw, ((0, 0), (0, 10))).reshape(B * 16)

    scan, merge = _build()
    cand = scan(predt, praw)
    out16 = merge(praw, cand, jnp.asarray(_TBL))
    return (out16[:4], out16[4], out16[5], out16[6])
